# baseline (speedup 1.0000x reference)
import functools

import jax
import jax.numpy as jnp
from jax import lax
from jax.experimental import pallas as pl
from jax.experimental.pallas import tpu as pltpu
from jax.experimental.pallas import tpu_sc as plsc

N = 10000
E = 320000
D = 128
H = 128
OUT = 128
L = 5
B = 64

NC = 2
NS = 16
NW = NC * NS
CH = 64
ER = 5120
RPW = ER // NW
NBUF = 4
TSTEPS = RPW // NBUF
NP = 10240
RPT = NP // NS
ZR = 16

_mesh = plsc.VectorSubcoreMesh(core_axis_name="c", subcore_axis_name="s")


@functools.partial(
    pl.kernel,
    mesh=_mesh,
    out_type=jax.ShapeDtypeStruct((NC, NP, D), jnp.float32),
    scratch_types=[
        pltpu.VMEM((RPW // 2, 2 * CH), jnp.int32),
        pltpu.VMEM((NBUF, CH), jnp.int32),
        pltpu.VMEM((NBUF, CH, D), jnp.float32),
        pltpu.VMEM_SHARED((NP, D), jnp.float32),
        pltpu.SemaphoreType.DMA,
        pltpu.SemaphoreType.DMA,
        pltpu.SemaphoreType.DMA,
        pltpu.SemaphoreType.DMA,
        pltpu.SemaphoreType.DMA,
        pltpu.SemaphoreType.DMA,
        pltpu.SemaphoreType.DMA,
        pltpu.SemaphoreType.DMA,
        pltpu.SemaphoreType.DMA,
        pltpu.SemaphoreType.DMA,
        pltpu.SemaphoreType.DMA,
        pltpu.SemaphoreType.DMA,
    ],
)
def _spmm(h_hbm, src_hbm, dst_hbm, z_hbm, out_hbm,
          idxd, isrc, rows, agg,
          g0, g1, g2, g3, s0, s1, s2, s3, i0, i1, i2, i3):
    c = lax.axis_index("c")
    s = lax.axis_index("s")
    wid = c * NS + s
    row0 = s * RPT
    wrow = wid * RPW

    gs = (g0, g1, g2, g3)
    ss = (s0, s1, s2, s3)
    isem = (i0, i1, i2, i3)

    for k in range(CH // ZR):
        pltpu.async_copy(z_hbm, rows.at[0, pl.ds(k * ZR, ZR), :], g0)
    for k in range(CH // ZR):
        pltpu.make_async_copy(z_hbm, rows.at[0, pl.ds(k * ZR, ZR), :],
                              g0).wait()
    for k in range(RPT // CH):
        pltpu.async_copy(rows.at[0], agg.at[pl.ds(row0 + k * CH, CH), :], g1)
    pltpu.async_copy(dst_hbm.at[pl.ds(wid * (RPW // 2), RPW // 2), :], idxd,
                     i0)
    for k in range(RPT // CH):
        pltpu.make_async_copy(rows.at[0], agg.at[pl.ds(row0 + k * CH, CH), :],
                              g1).wait()
    pltpu.make_async_copy(dst_hbm.at[pl.ds(wid * (RPW // 2), RPW // 2), :],
                          idxd, i0).wait()
    plsc.subcore_barrier()

    def gather_start(r2, hh, b):
        pltpu.async_copy(h_hbm.at[idxd.at[r2, pl.ds(hh * CH, CH)]],
                         rows.at[b], gs[b])

    def gather_wait(r2, hh, b):
        pltpu.make_async_copy(h_hbm.at[idxd.at[r2, pl.ds(hh * CH, CH)]],
                              rows.at[b], gs[b]).wait()

    def isrc_start(r, b):
        pltpu.async_copy(src_hbm.at[wrow + r], isrc.at[b], isem[b])

    def isrc_wait(r, b):
        pltpu.make_async_copy(src_hbm.at[wrow + r], isrc.at[b], isem[b]).wait()

    def scat_start(r, b):
        pltpu.async_copy(rows.at[b], agg.at[isrc.at[b]], ss[b], add=True)

    def scat_wait(r, b):
        pltpu.make_async_copy(rows.at[b], agg.at[isrc.at[b]], ss[b]).wait()

    for b in (0, 1):
        isrc_start(b, b)
        gather_start(0, b, b)

    def outer(t, carry):
        for b in range(NBUF):
            r = NBUF * t + b
            r2 = 2 * t + b // 2
            pb = (b + 2) % NBUF
            p = r + 2
            p2 = 2 * t + (b + 2) // 2
            gather_wait(r2, b % 2, b)
            isrc_wait(r, b)
            scat_start(r, b)

            def prefetch():
                scat_wait(p - NBUF, pb)
                isrc_start(p, pb)
                gather_start(p2, b % 2, pb)

            if b < 2:
                pl.when(t >= 1)(prefetch)

                def prefetch0():
                    isrc_start(p, pb)
                    gather_start(p2, b % 2, pb)

                pl.when(t < 1)(prefetch0)
            else:
                pl.when(t < TSTEPS - 1)(prefetch)
        return carry

    lax.fori_loop(0, TSTEPS, outer, 0)

    for b in range(NBUF):
        scat_wait(RPW - NBUF + b, b)

    plsc.subcore_barrier()
    pltpu.sync_copy(agg.at[pl.ds(row0, RPT), :],
                    out_hbm.at[c, pl.ds(row0, RPT), :])


def _mlp_body(eps_ref, a, h, w1, b1, g1, be1, w2, b2, g2, be2, o):
    agg = a[0, :N, :] + a[1, :N, :] + eps_ref[0, 0] * h[...]
    z = jnp.dot(agg, w1[...], preferred_element_type=jnp.float32) + b1[...]
    mu = jnp.mean(z, axis=0, keepdims=True)
    var = jnp.mean((z - mu) ** 2, axis=0, keepdims=True)
    z = jnp.maximum((z - mu) * lax.rsqrt(var + 1e-5) * g1[...] + be1[...], 0.0)
    z = jnp.dot(z, w2[...], preferred_element_type=jnp.float32) + b2[...]
    mu = jnp.mean(z, axis=0, keepdims=True)
    var = jnp.mean((z - mu) ** 2, axis=0, keepdims=True)
    o[...] = jnp.maximum((z - mu) * lax.rsqrt(var + 1e-5) * g2[...] + be2[...], 0.0)


_mlp = pl.pallas_call(
    _mlp_body,
    out_shape=jax.ShapeDtypeStruct((N, H), jnp.float32),
    in_specs=[pl.BlockSpec(memory_space=pltpu.SMEM)]
    + [pl.BlockSpec(memory_space=pltpu.VMEM)] * 10,
    out_specs=pl.BlockSpec(memory_space=pltpu.VMEM),
)


def _pool_body(gid, h0, h1, h2, h3, h4, w0, w1, w2, w3, w4,
               c0, c1, c2, c3, c4, o):
    ids = gid[...]
    onehot = jnp.where(
        ids == lax.broadcasted_iota(jnp.int32, (1, B), 1), 1.0, 0.0)
    acc = c0[...] + c1[...] + c2[...] + c3[...] + c4[...]
    for hh, ww in ((h0, w0), (h1, w1), (h2, w2), (h3, w3), (h4, w4)):
        pooled = lax.dot_general(onehot, hh[...], (((0,), (0,)), ((), ())),
                                 preferred_element_type=jnp.float32)
        acc = acc + jnp.dot(pooled, ww[...],
                            preferred_element_type=jnp.float32)
    o[...] = acc


_pool = pl.pallas_call(
    _pool_body,
    out_shape=jax.ShapeDtypeStruct((B, OUT), jnp.float32),
    in_specs=[pl.BlockSpec(memory_space=pltpu.VMEM)] * 16,
    out_specs=pl.BlockSpec(memory_space=pltpu.VMEM),
)


def kernel(x, edge_index, graph_ids, eps, mlp_params, bn_params, lin_params):
    src = edge_index[0]
    dst = edge_index[1]
    npad = ER * CH - E
    pad_src = (N + (jnp.arange(npad, dtype=jnp.int32) % (NP - N))).reshape(
        -1, CH)
    pad_dst = (jnp.arange(npad, dtype=jnp.int32) % N).reshape(-1, 2 * CH)
    src2d = jnp.concatenate([src.reshape(-1, CH), pad_src], axis=0)
    dst2d = jnp.concatenate([dst.reshape(-1, 2 * CH), pad_dst], axis=0)
    zeros = jnp.zeros((ZR, D), jnp.float32)

    h = x
    hiddens = [x]
    for l in range(L - 1):
        parts = _spmm(h, src2d, dst2d, zeros)
        p = mlp_params[l]
        g2, be2 = bn_params[l]
        epsp = (1.0 + eps[l]).reshape(1, 1)
        h = _mlp(epsp, parts, h,
                 p['W1'], p['b1'].reshape(1, H), p['g1'].reshape(1, H),
                 p['be1'].reshape(1, H),
                 p['W2'], p['b2'].reshape(1, H), g2.reshape(1, H),
                 be2.reshape(1, H))
        hiddens.append(h)

    ws = [w for w, _ in lin_params]
    bs = [b.reshape(1, OUT) for _, b in lin_params]
    score = _pool(graph_ids.reshape(N, 1), *hiddens, *ws, *bs)
    return score


# speedup vs baseline: 1.0031x; 1.0031x over previous
import functools

import jax
import jax.numpy as jnp
from jax import lax
from jax.experimental import pallas as pl
from jax.experimental.pallas import tpu as pltpu
from jax.experimental.pallas import tpu_sc as plsc

N = 10000
E = 320000
D = 128
H = 128
OUT = 128
L = 5
B = 64

NC = 2
NS = 16
NW = NC * NS
CH = 64
ER = 5120
RPW = ER // NW
NBUF = 4
TSTEPS = RPW // NBUF
NP = 10240
RPT = NP // NS
ZR = 16

_mesh = plsc.VectorSubcoreMesh(core_axis_name="c", subcore_axis_name="s")


@functools.partial(
    pl.kernel,
    mesh=_mesh,
    out_type=jax.ShapeDtypeStruct((NC, NP, D), jnp.float32),
    scratch_types=[
        pltpu.VMEM((RPW // 2, 2 * CH), jnp.int32),
        pltpu.VMEM((NBUF, CH), jnp.int32),
        pltpu.VMEM((NBUF, CH, D), jnp.float32),
        pltpu.VMEM_SHARED((NP, D), jnp.float32),
        pltpu.SemaphoreType.DMA,
        pltpu.SemaphoreType.DMA,
        pltpu.SemaphoreType.DMA,
        pltpu.SemaphoreType.DMA,
        pltpu.SemaphoreType.DMA,
        pltpu.SemaphoreType.DMA,
        pltpu.SemaphoreType.DMA,
        pltpu.SemaphoreType.DMA,
        pltpu.SemaphoreType.DMA,
        pltpu.SemaphoreType.DMA,
        pltpu.SemaphoreType.DMA,
        pltpu.SemaphoreType.DMA,
    ],
)
def _spmm(h_hbm, src_hbm, dst_hbm, z_hbm, out_hbm,
          idxd, isrc, rows, agg,
          g0, g1, g2, g3, s0, s1, s2, s3, i0, i1, i2, i3):
    c = lax.axis_index("c")
    s = lax.axis_index("s")
    wid = c * NS + s
    row0 = s * RPT
    wrow = wid * RPW

    gs = (g0, g1, g2, g3)
    ss = (s0, s1, s2, s3)
    isem = (i0, i1, i2, i3)

    for k in range(CH // ZR):
        pltpu.async_copy(z_hbm, rows.at[0, pl.ds(k * ZR, ZR), :], g0)
    for k in range(CH // ZR):
        pltpu.make_async_copy(z_hbm, rows.at[0, pl.ds(k * ZR, ZR), :],
                              g0).wait()
    for k in range(RPT // CH):
        pltpu.async_copy(rows.at[0], agg.at[pl.ds(row0 + k * CH, CH), :], g1)
    pltpu.async_copy(dst_hbm.at[pl.ds(wid * (RPW // 2), RPW // 2), :], idxd,
                     i0)
    for k in range(RPT // CH):
        pltpu.make_async_copy(rows.at[0], agg.at[pl.ds(row0 + k * CH, CH), :],
                              g1).wait()
    pltpu.make_async_copy(dst_hbm.at[pl.ds(wid * (RPW // 2), RPW // 2), :],
                          idxd, i0).wait()
    plsc.subcore_barrier()

    def gather_start(r2, hh, b):
        pltpu.async_copy(h_hbm.at[idxd.at[r2, pl.ds(hh * CH, CH)]],
                         rows.at[b], gs[b])

    def gather_wait(r2, hh, b):
        pltpu.make_async_copy(h_hbm.at[idxd.at[r2, pl.ds(hh * CH, CH)]],
                              rows.at[b], gs[b]).wait()

    def isrc_start(r, b):
        pltpu.async_copy(src_hbm.at[wrow + r], isrc.at[b], isem[b])

    def isrc_wait(r, b):
        pltpu.make_async_copy(src_hbm.at[wrow + r], isrc.at[b], isem[b]).wait()

    def scat_start(r, b):
        pltpu.async_copy(rows.at[b], agg.at[isrc.at[b]], ss[b], add=True)

    def scat_wait(r, b):
        pltpu.make_async_copy(rows.at[b], agg.at[isrc.at[b]], ss[b]).wait()

    for b in (0, 1):
        isrc_start(b, b)
        gather_start(0, b, b)

    def outer(t, carry):
        for b in range(NBUF):
            r = NBUF * t + b
            r2 = 2 * t + b // 2
            pb = (b + 2) % NBUF
            p = r + 2
            p2 = 2 * t + (b + 2) // 2
            gather_wait(r2, b % 2, b)
            isrc_wait(r, b)
            scat_start(r, b)

            def prefetch():
                scat_wait(p - NBUF, pb)
                isrc_start(p, pb)
                gather_start(p2, b % 2, pb)

            if b < 2:
                pl.when(t >= 1)(prefetch)

                def prefetch0():
                    isrc_start(p, pb)
                    gather_start(p2, b % 2, pb)

                pl.when(t < 1)(prefetch0)
            else:
                pl.when(t < TSTEPS - 1)(prefetch)
        return carry

    lax.fori_loop(0, TSTEPS, outer, 0)

    for b in range(NBUF):
        scat_wait(RPW - NBUF + b, b)

    plsc.subcore_barrier()
    pltpu.sync_copy(agg.at[pl.ds(row0, RPT), :],
                    out_hbm.at[c, pl.ds(row0, RPT), :])


def _onehot(gid):
    return jnp.where(
        gid[...] == lax.broadcasted_iota(jnp.int32, (1, B), 1), 1.0, 0.0)


def _mlp_body(eps_ref, a, h, gid, w1, b1, g1, be1, w2, b2, g2, be2, wl,
              o, po):
    agg = a[0, :N, :] + a[1, :N, :] + eps_ref[0, 0] * h[...]
    z = jnp.dot(agg, w1[...], preferred_element_type=jnp.float32) + b1[...]
    mu = jnp.mean(z, axis=0, keepdims=True)
    var = jnp.mean((z - mu) ** 2, axis=0, keepdims=True)
    z = jnp.maximum((z - mu) * lax.rsqrt(var + 1e-5) * g1[...] + be1[...], 0.0)
    z = jnp.dot(z, w2[...], preferred_element_type=jnp.float32) + b2[...]
    mu = jnp.mean(z, axis=0, keepdims=True)
    var = jnp.mean((z - mu) ** 2, axis=0, keepdims=True)
    res = jnp.maximum(
        (z - mu) * lax.rsqrt(var + 1e-5) * g2[...] + be2[...], 0.0)
    o[...] = res
    pooled = lax.dot_general(_onehot(gid), res, (((0,), (0,)), ((), ())),
                             preferred_element_type=jnp.float32)
    po[...] = jnp.dot(pooled, wl[...], preferred_element_type=jnp.float32)


_mlp = pl.pallas_call(
    _mlp_body,
    out_shape=(jax.ShapeDtypeStruct((N, H), jnp.float32),
               jax.ShapeDtypeStruct((B, OUT), jnp.float32)),
    in_specs=[pl.BlockSpec(memory_space=pltpu.SMEM)]
    + [pl.BlockSpec(memory_space=pltpu.VMEM)] * 12,
    out_specs=(pl.BlockSpec(memory_space=pltpu.VMEM),
               pl.BlockSpec(memory_space=pltpu.VMEM)),
)


def _pool_body(gid, hx, w0, p1, p2, p3, p4, c0, c1, c2, c3, c4, o):
    pooled = lax.dot_general(_onehot(gid), hx[...], (((0,), (0,)), ((), ())),
                             preferred_element_type=jnp.float32)
    o[...] = (jnp.dot(pooled, w0[...], preferred_element_type=jnp.float32)
              + p1[...] + p2[...] + p3[...] + p4[...]
              + c0[...] + c1[...] + c2[...] + c3[...] + c4[...])


_pool = pl.pallas_call(
    _pool_body,
    out_shape=jax.ShapeDtypeStruct((B, OUT), jnp.float32),
    in_specs=[pl.BlockSpec(memory_space=pltpu.VMEM)] * 12,
    out_specs=pl.BlockSpec(memory_space=pltpu.VMEM),
)


def kernel(x, edge_index, graph_ids, eps, mlp_params, bn_params, lin_params):
    src = edge_index[0]
    dst = edge_index[1]
    npad = ER * CH - E
    pad_src = (N + (jnp.arange(npad, dtype=jnp.int32) % (NP - N))).reshape(
        -1, CH)
    pad_dst = (jnp.arange(npad, dtype=jnp.int32) % N).reshape(-1, 2 * CH)
    src2d = jnp.concatenate([src.reshape(-1, CH), pad_src], axis=0)
    dst2d = jnp.concatenate([dst.reshape(-1, 2 * CH), pad_dst], axis=0)
    zeros = jnp.zeros((ZR, D), jnp.float32)

    ws = [w for w, _ in lin_params]
    bs = [b.reshape(1, OUT) for _, b in lin_params]
    gid = graph_ids.reshape(N, 1)

    h = x
    pools = []
    for l in range(L - 1):
        parts = _spmm(h, src2d, dst2d, zeros)
        p = mlp_params[l]
        g2, be2 = bn_params[l]
        epsp = (1.0 + eps[l]).reshape(1, 1)
        h, pl_contrib = _mlp(epsp, parts, h, gid,
                             p['W1'], p['b1'].reshape(1, H),
                             p['g1'].reshape(1, H), p['be1'].reshape(1, H),
                             p['W2'], p['b2'].reshape(1, H),
                             g2.reshape(1, H), be2.reshape(1, H), ws[l + 1])
        pools.append(pl_contrib)

    score = _pool(gid, x, ws[0], *pools, *bs)
    return score


# speedup vs baseline: 1.1261x; 1.1226x over previous
import functools

import jax
import jax.numpy as jnp
from jax import lax
from jax.experimental import pallas as pl
from jax.experimental.pallas import tpu as pltpu
from jax.experimental.pallas import tpu_sc as plsc

N = 10000
E = 320000
D = 128
H = 128
OUT = 128
L = 5
B = 64

NC = 2
NS = 16
NW = NC * NS
CH = 64
ER = 5120
RPW = ER // NW
NBUF = 4
TSTEPS = RPW // NBUF
NP = 10240
RPT = NP // NS
ZR = 16

_mesh = plsc.VectorSubcoreMesh(core_axis_name="c", subcore_axis_name="s")


@functools.partial(
    pl.kernel,
    mesh=_mesh,
    out_type=jax.ShapeDtypeStruct((NC, NP, D), jnp.float32),
    scratch_types=[
        pltpu.VMEM((RPW // 2, 2 * CH), jnp.int32),
        pltpu.VMEM((NBUF, CH), jnp.int32),
        pltpu.VMEM((NBUF, CH, D), jnp.float32),
        pltpu.VMEM_SHARED((NP, D), jnp.float32),
        pltpu.SemaphoreType.DMA,
        pltpu.SemaphoreType.DMA,
        pltpu.SemaphoreType.DMA,
        pltpu.SemaphoreType.DMA,
        pltpu.SemaphoreType.DMA,
        pltpu.SemaphoreType.DMA,
        pltpu.SemaphoreType.DMA,
        pltpu.SemaphoreType.DMA,
        pltpu.SemaphoreType.DMA,
        pltpu.SemaphoreType.DMA,
        pltpu.SemaphoreType.DMA,
        pltpu.SemaphoreType.DMA,
    ],
)
def _spmm(h_hbm, src_hbm, dst_hbm, z_hbm, out_hbm,
          idxd, isrc, rows, agg,
          g0, g1, g2, g3, s0, s1, s2, s3, i0, i1, i2, i3):
    c = lax.axis_index("c")
    s = lax.axis_index("s")
    wid = c * NS + s
    row0 = s * RPT
    wrow = wid * RPW

    gs = (g0, g1, g2, g3)
    ss = (s0, s1, s2, s3)
    isem = (i0, i1, i2, i3)

    for k in range(CH // ZR):
        pltpu.async_copy(z_hbm, rows.at[0, pl.ds(k * ZR, ZR), :], g0)
    for k in range(CH // ZR):
        pltpu.make_async_copy(z_hbm, rows.at[0, pl.ds(k * ZR, ZR), :],
                              g0).wait()
    for k in range(RPT // CH):
        pltpu.async_copy(rows.at[0], agg.at[pl.ds(row0 + k * CH, CH), :], g1)
    pltpu.async_copy(dst_hbm.at[pl.ds(wid * (RPW // 2), RPW // 2), :], idxd,
                     i0)
    for k in range(RPT // CH):
        pltpu.make_async_copy(rows.at[0], agg.at[pl.ds(row0 + k * CH, CH), :],
                              g1).wait()
    pltpu.make_async_copy(dst_hbm.at[pl.ds(wid * (RPW // 2), RPW // 2), :],
                          idxd, i0).wait()
    plsc.subcore_barrier()

    def gather_start(r2, hh, b):
        pltpu.async_copy(h_hbm.at[idxd.at[r2, pl.ds(hh * CH, CH)]],
                         rows.at[b], gs[b])

    def gather_wait(r2, hh, b):
        pltpu.make_async_copy(h_hbm.at[idxd.at[r2, pl.ds(hh * CH, CH)]],
                              rows.at[b], gs[b]).wait()

    def isrc_start(r, b):
        pltpu.async_copy(src_hbm.at[wrow + r], isrc.at[b], isem[b])

    def isrc_wait(r, b):
        pltpu.make_async_copy(src_hbm.at[wrow + r], isrc.at[b], isem[b]).wait()

    def scat_start(r, b):
        pltpu.async_copy(rows.at[b], agg.at[isrc.at[b]], ss[b], add=True)

    def scat_wait(r, b):
        pltpu.make_async_copy(rows.at[b], agg.at[isrc.at[b]], ss[b]).wait()

    isrc_start(0, 0)
    gather_start(0, 0, 0)
    isrc_start(1, 1)
    gather_start(0, 1, 1)
    isrc_start(2, 2)
    gather_start(1, 0, 2)

    def outer(t, carry):
        for b in range(NBUF):
            r = NBUF * t + b
            r2 = 2 * t + b // 2
            pb = (b + 3) % NBUF
            p = r + 3
            p2 = (4 * t + b + 3) // 2
            ph = (b + 1) % 2
            gather_wait(r2, b % 2, b)
            isrc_wait(r, b)
            scat_start(r, b)

            def prefetch():
                scat_wait(p - NBUF, pb)
                isrc_start(p, pb)
                gather_start(p2, ph, pb)

            if b == 0:
                pl.when(t >= 1)(prefetch)

                def prefetch0():
                    isrc_start(p, pb)
                    gather_start(p2, ph, pb)

                pl.when(t < 1)(prefetch0)
            else:
                pl.when(t < TSTEPS - 1)(prefetch)
        return carry

    lax.fori_loop(0, TSTEPS, outer, 0)

    for b in range(NBUF):
        scat_wait(RPW - NBUF + b, b)

    plsc.subcore_barrier()
    pltpu.sync_copy(agg.at[pl.ds(row0, RPT), :],
                    out_hbm.at[c, pl.ds(row0, RPT), :])


def _onehot(gid):
    return jnp.where(
        gid[...] == lax.broadcasted_iota(jnp.int32, (1, B), 1), 1.0, 0.0)


def _mlp_body(eps_ref, a, h, gid, w1, b1, g1, be1, w2, b2, g2, be2, wl,
              o, po):
    agg = a[0, :N, :] + a[1, :N, :] + eps_ref[0, 0] * h[...]
    z = jnp.dot(agg, w1[...], preferred_element_type=jnp.float32) + b1[...]
    mu = jnp.mean(z, axis=0, keepdims=True)
    var = jnp.mean((z - mu) ** 2, axis=0, keepdims=True)
    z = jnp.maximum((z - mu) * lax.rsqrt(var + 1e-5) * g1[...] + be1[...], 0.0)
    z = jnp.dot(z, w2[...], preferred_element_type=jnp.float32) + b2[...]
    mu = jnp.mean(z, axis=0, keepdims=True)
    var = jnp.mean((z - mu) ** 2, axis=0, keepdims=True)
    res = jnp.maximum(
        (z - mu) * lax.rsqrt(var + 1e-5) * g2[...] + be2[...], 0.0)
    o[...] = res
    pooled = lax.dot_general(_onehot(gid), res, (((0,), (0,)), ((), ())),
                             preferred_element_type=jnp.float32)
    po[...] = jnp.dot(pooled, wl[...], preferred_element_type=jnp.float32)


_mlp = pl.pallas_call(
    _mlp_body,
    out_shape=(jax.ShapeDtypeStruct((N, H), jnp.float32),
               jax.ShapeDtypeStruct((B, OUT), jnp.float32)),
    in_specs=[pl.BlockSpec(memory_space=pltpu.SMEM)]
    + [pl.BlockSpec(memory_space=pltpu.VMEM)] * 12,
    out_specs=(pl.BlockSpec(memory_space=pltpu.VMEM),
               pl.BlockSpec(memory_space=pltpu.VMEM)),
)


def _pool_body(gid, hx, w0, p1, p2, p3, p4, c0, c1, c2, c3, c4, o):
    pooled = lax.dot_general(_onehot(gid), hx[...], (((0,), (0,)), ((), ())),
                             preferred_element_type=jnp.float32)
    o[...] = (jnp.dot(pooled, w0[...], preferred_element_type=jnp.float32)
              + p1[...] + p2[...] + p3[...] + p4[...]
              + c0[...] + c1[...] + c2[...] + c3[...] + c4[...])


_pool = pl.pallas_call(
    _pool_body,
    out_shape=jax.ShapeDtypeStruct((B, OUT), jnp.float32),
    in_specs=[pl.BlockSpec(memory_space=pltpu.VMEM)] * 12,
    out_specs=pl.BlockSpec(memory_space=pltpu.VMEM),
)


def kernel(x, edge_index, graph_ids, eps, mlp_params, bn_params, lin_params):
    src = edge_index[0]
    dst = edge_index[1]
    npad = ER * CH - E
    pad_src = (N + (jnp.arange(npad, dtype=jnp.int32) % (NP - N))).reshape(
        -1, CH)
    pad_dst = (jnp.arange(npad, dtype=jnp.int32) % N).reshape(-1, 2 * CH)
    src2d = jnp.concatenate([src.reshape(-1, CH), pad_src], axis=0)
    dst2d = jnp.concatenate([dst.reshape(-1, 2 * CH), pad_dst], axis=0)
    zeros = jnp.zeros((ZR, D), jnp.float32)

    ws = [w for w, _ in lin_params]
    bs = [b.reshape(1, OUT) for _, b in lin_params]
    gid = graph_ids.reshape(N, 1)

    h = x
    pools = []
    for l in range(L - 1):
        parts = _spmm(h, src2d, dst2d, zeros)
        p = mlp_params[l]
        g2, be2 = bn_params[l]
        epsp = (1.0 + eps[l]).reshape(1, 1)
        h, pl_contrib = _mlp(epsp, parts, h, gid,
                             p['W1'], p['b1'].reshape(1, H),
                             p['g1'].reshape(1, H), p['be1'].reshape(1, H),
                             p['W2'], p['b2'].reshape(1, H),
                             g2.reshape(1, H), be2.reshape(1, H), ws[l + 1])
        pools.append(pl_contrib)

    score = _pool(gid, x, ws[0], *pools, *bs)
    return score


# speedup vs baseline: 1.1831x; 1.0506x over previous
import functools

import jax
import jax.numpy as jnp
from jax import lax
from jax.experimental import pallas as pl
from jax.experimental.pallas import tpu as pltpu
from jax.experimental.pallas import tpu_sc as plsc

N = 10000
E = 320000
D = 128
H = 128
OUT = 128
L = 5
B = 64

NC = 2
NS = 16
NW = NC * NS
CH = 32
CPR = 128 // CH
ER = 10240
RPW = ER // NW
NBUF = 8
DEPTH = 6
TSTEPS = RPW // NBUF
NP = 10240
RPT = NP // NS
ZR = 16

_mesh = plsc.VectorSubcoreMesh(core_axis_name="c", subcore_axis_name="s")


@functools.partial(
    pl.kernel,
    mesh=_mesh,
    out_type=jax.ShapeDtypeStruct((NC, NP, D), jnp.float32),
    scratch_types=[
        pltpu.VMEM((RPW // CPR, CPR * CH), jnp.int32),
        pltpu.VMEM((NBUF, CH), jnp.int32),
        pltpu.VMEM((NBUF, CH, D), jnp.float32),
        pltpu.VMEM_SHARED((NP, D), jnp.float32),
        pltpu.SemaphoreType.DMA,
        pltpu.SemaphoreType.DMA,
        pltpu.SemaphoreType.DMA,
        pltpu.SemaphoreType.DMA,
        pltpu.SemaphoreType.DMA,
        pltpu.SemaphoreType.DMA,
        pltpu.SemaphoreType.DMA,
        pltpu.SemaphoreType.DMA,
        pltpu.SemaphoreType.DMA,
        pltpu.SemaphoreType.DMA,
        pltpu.SemaphoreType.DMA,
        pltpu.SemaphoreType.DMA,
        pltpu.SemaphoreType.DMA,
        pltpu.SemaphoreType.DMA,
        pltpu.SemaphoreType.DMA,
        pltpu.SemaphoreType.DMA,
        pltpu.SemaphoreType.DMA,
        pltpu.SemaphoreType.DMA,
        pltpu.SemaphoreType.DMA,
        pltpu.SemaphoreType.DMA,
        pltpu.SemaphoreType.DMA,
        pltpu.SemaphoreType.DMA,
        pltpu.SemaphoreType.DMA,
        pltpu.SemaphoreType.DMA,
    ],
)
def _spmm(h_hbm, src_hbm, dst_hbm, z_hbm, out_hbm,
          idxd, isrc, rows, agg, *sems):
    c = lax.axis_index("c")
    s = lax.axis_index("s")
    wid = c * NS + s
    row0 = s * RPT
    wrow = wid * RPW

    gs = sems[0:NBUF]
    ss = sems[NBUF:2 * NBUF]
    isem = sems[2 * NBUF:3 * NBUF]
    g0, g1, i0 = gs[0], gs[1], isem[0]

    for k in range(CH // ZR):
        pltpu.async_copy(z_hbm, rows.at[0, pl.ds(k * ZR, ZR), :], g0)
    for k in range(CH // ZR):
        pltpu.make_async_copy(z_hbm, rows.at[0, pl.ds(k * ZR, ZR), :],
                              g0).wait()
    for k in range(RPT // CH):
        pltpu.async_copy(rows.at[0], agg.at[pl.ds(row0 + k * CH, CH), :], g1)
    pltpu.async_copy(dst_hbm.at[pl.ds(wid * (RPW // CPR), RPW // CPR), :],
                     idxd, i0)
    for k in range(RPT // CH):
        pltpu.make_async_copy(rows.at[0], agg.at[pl.ds(row0 + k * CH, CH), :],
                              g1).wait()
    pltpu.make_async_copy(dst_hbm.at[pl.ds(wid * (RPW // CPR), RPW // CPR), :],
                          idxd, i0).wait()
    plsc.subcore_barrier()

    def gather_start(r, b):
        pltpu.async_copy(
            h_hbm.at[idxd.at[r // CPR, pl.ds((r % CPR) * CH, CH)]],
            rows.at[b], gs[b])

    def gather_wait(r, b):
        pltpu.make_async_copy(
            h_hbm.at[idxd.at[r // CPR, pl.ds((r % CPR) * CH, CH)]],
            rows.at[b], gs[b]).wait()

    def isrc_start(r, b):
        pltpu.async_copy(src_hbm.at[wrow + r], isrc.at[b], isem[b])

    def isrc_wait(r, b):
        pltpu.make_async_copy(src_hbm.at[wrow + r], isrc.at[b], isem[b]).wait()

    def scat_start(r, b):
        pltpu.async_copy(rows.at[b], agg.at[isrc.at[b]], ss[b], add=True)

    def scat_wait(r, b):
        pltpu.make_async_copy(rows.at[b], agg.at[isrc.at[b]], ss[b]).wait()

    for j in range(DEPTH):
        isrc_start(j, j)
        gather_start(j, j)

    def outer(t, carry):
        for b in range(NBUF):
            r = NBUF * t + b
            pb = (b + DEPTH) % NBUF
            p = r + DEPTH
            gather_wait(r, b)
            isrc_wait(r, b)
            scat_start(r, b)

            def prefetch():
                scat_wait(p - NBUF, pb)
                isrc_start(p, pb)
                gather_start(p, pb)

            if b < NBUF - DEPTH:
                pl.when(t >= 1)(prefetch)

                def prefetch0():
                    isrc_start(p, pb)
                    gather_start(p, pb)

                pl.when(t < 1)(prefetch0)
            else:
                pl.when(t < TSTEPS - 1)(prefetch)
        return carry

    lax.fori_loop(0, TSTEPS, outer, 0)

    for b in range(NBUF):
        scat_wait(RPW - NBUF + b, b)

    plsc.subcore_barrier()
    pltpu.sync_copy(agg.at[pl.ds(row0, RPT), :],
                    out_hbm.at[c, pl.ds(row0, RPT), :])


def _onehot(gid):
    return jnp.where(
        gid[...] == lax.broadcasted_iota(jnp.int32, (1, B), 1), 1.0, 0.0)


def _mlp_body(eps_ref, a, h, gid, w1, b1, g1, be1, w2, b2, g2, be2, wl,
              o, po):
    agg = a[0, :N, :] + a[1, :N, :] + eps_ref[0, 0] * h[...]
    z = jnp.dot(agg, w1[...], preferred_element_type=jnp.float32) + b1[...]
    mu = jnp.mean(z, axis=0, keepdims=True)
    var = jnp.mean((z - mu) ** 2, axis=0, keepdims=True)
    z = jnp.maximum((z - mu) * lax.rsqrt(var + 1e-5) * g1[...] + be1[...], 0.0)
    z = jnp.dot(z, w2[...], preferred_element_type=jnp.float32) + b2[...]
    mu = jnp.mean(z, axis=0, keepdims=True)
    var = jnp.mean((z - mu) ** 2, axis=0, keepdims=True)
    res = jnp.maximum(
        (z - mu) * lax.rsqrt(var + 1e-5) * g2[...] + be2[...], 0.0)
    o[...] = res
    pooled = lax.dot_general(_onehot(gid), res, (((0,), (0,)), ((), ())),
                             preferred_element_type=jnp.float32)
    po[...] = jnp.dot(pooled, wl[...], preferred_element_type=jnp.float32)


_mlp = pl.pallas_call(
    _mlp_body,
    out_shape=(jax.ShapeDtypeStruct((N, H), jnp.float32),
               jax.ShapeDtypeStruct((B, OUT), jnp.float32)),
    in_specs=[pl.BlockSpec(memory_space=pltpu.SMEM)]
    + [pl.BlockSpec(memory_space=pltpu.VMEM)] * 12,
    out_specs=(pl.BlockSpec(memory_space=pltpu.VMEM),
               pl.BlockSpec(memory_space=pltpu.VMEM)),
)


def _pool_body(gid, hx, w0, p1, p2, p3, p4, c0, c1, c2, c3, c4, o):
    pooled = lax.dot_general(_onehot(gid), hx[...], (((0,), (0,)), ((), ())),
                             preferred_element_type=jnp.float32)
    o[...] = (jnp.dot(pooled, w0[...], preferred_element_type=jnp.float32)
              + p1[...] + p2[...] + p3[...] + p4[...]
              + c0[...] + c1[...] + c2[...] + c3[...] + c4[...])


_pool = pl.pallas_call(
    _pool_body,
    out_shape=jax.ShapeDtypeStruct((B, OUT), jnp.float32),
    in_specs=[pl.BlockSpec(memory_space=pltpu.VMEM)] * 12,
    out_specs=pl.BlockSpec(memory_space=pltpu.VMEM),
)


def kernel(x, edge_index, graph_ids, eps, mlp_params, bn_params, lin_params):
    src = edge_index[0]
    dst = edge_index[1]
    npad = ER * CH - E
    pad_src = (N + (jnp.arange(npad, dtype=jnp.int32) % (NP - N))).reshape(
        -1, CH)
    pad_dst = (jnp.arange(npad, dtype=jnp.int32) % N).reshape(-1, CPR * CH)
    src2d = jnp.concatenate([src.reshape(-1, CH), pad_src], axis=0)
    dst2d = jnp.concatenate([dst.reshape(-1, CPR * CH), pad_dst], axis=0)
    zeros = jnp.zeros((ZR, D), jnp.float32)

    ws = [w for w, _ in lin_params]
    bs = [b.reshape(1, OUT) for _, b in lin_params]
    gid = graph_ids.reshape(N, 1)

    h = x
    pools = []
    for l in range(L - 1):
        parts = _spmm(h, src2d, dst2d, zeros)
        p = mlp_params[l]
        g2, be2 = bn_params[l]
        epsp = (1.0 + eps[l]).reshape(1, 1)
        h, pl_contrib = _mlp(epsp, parts, h, gid,
                             p['W1'], p['b1'].reshape(1, H),
                             p['g1'].reshape(1, H), p['be1'].reshape(1, H),
                             p['W2'], p['b2'].reshape(1, H),
                             g2.reshape(1, H), be2.reshape(1, H), ws[l + 1])
        pools.append(pl_contrib)

    score = _pool(gid, x, ws[0], *pools, *bs)
    return score
